# Initial kernel scaffold; baseline (speedup 1.0000x reference)
import jax
import jax.numpy as jnp
from jax.experimental import pallas as pl


def kernel(scores, labels, property_ids):
    raise NotImplementedError("write your pallas kernel here")


# speedup vs baseline: 1.9112x; 1.9112x over previous
import functools

import jax
import jax.numpy as jnp
from jax import lax
from jax.experimental import pallas as pl
from jax.experimental.pallas import tpu as pltpu
from jax.experimental.pallas import tpu_sc as plsc

MARGIN = 1.0
N = 4096
NPROP = 128
L = 16
NC, NS = 2, 16
NW = NC * NS
PPW = NPROP // NW
NCHUNK = N // L
BUF = N + L
NEG_PAD = -1.0e30


def _sc_body(scores_hbm, labels_hbm, props_hbm, out_hbm,
             scores_v, labels_v, props_v, part_v, *bufs):
    pos_bufs = bufs[:PPW]
    neg_bufs = bufs[PPW:]
    wid = lax.axis_index("c") * NS + lax.axis_index("s")
    base_prop = wid * PPW

    pltpu.sync_copy(scores_hbm, scores_v)
    pltpu.sync_copy(labels_hbm, labels_v)
    pltpu.sync_copy(props_hbm, props_v)

    def chunk_body(k, offs):
        sl = pl.ds(k * L, L)
        s = scores_v[sl]
        is_pos = labels_v[sl] == 1
        p = props_v[sl]
        new_offs = [None] * (2 * PPW)
        for t in range(PPW):
            m_same = p == (base_prop + t)
            m_pos = m_same & is_pos
            m_neg = m_same & (~is_pos)
            cum_pos = plsc.cumsum(m_pos.astype(jnp.int32))
            cum_neg = plsc.cumsum(m_neg.astype(jnp.int32))
            idx_pos = offs[t] + jnp.maximum(cum_pos - 1, 0)
            idx_neg = offs[PPW + t] + jnp.maximum(cum_neg - 1, 0)
            plsc.store_scatter(pos_bufs[t], [idx_pos], s, mask=m_pos)
            plsc.store_scatter(neg_bufs[t], [idx_neg], s, mask=m_neg)
            new_offs[t] = offs[t] + cum_pos[L - 1]
            new_offs[PPW + t] = offs[PPW + t] + cum_neg[L - 1]
        return tuple(new_offs)

    zero = jnp.int32(0)
    counts = lax.fori_loop(0, NCHUNK, chunk_body, (zero,) * (2 * PPW))

    pad_vec = jnp.full((L,), NEG_PAD, jnp.float32)
    acc = jnp.zeros((L,), jnp.float32)
    pairs = zero
    for t in range(PPW):
        npos, nneg = counts[t], counts[PPW + t]
        neg_bufs[t][pl.ds(nneg, L)] = pad_vec
        pairs = pairs + npos * nneg
        nch = (nneg + (L - 1)) // L

        def pos_body(i, a, t=t, nch=nch):
            coef = MARGIN - pos_bufs[t][pl.ds(i, L)][0]

            def neg_body(c, aa, t=t, coef=coef):
                nv = neg_bufs[t][pl.ds(c * L, L)]
                return aa + jnp.maximum(coef + nv, 0.0)

            return lax.fori_loop(0, nch, neg_body, a)

        acc = lax.fori_loop(0, npos, pos_body, acc)

    loss = jnp.sum(acc)
    lane = lax.broadcasted_iota(jnp.int32, (L,), 0)
    part = jnp.where(lane == 0, loss,
                     jnp.where(lane == 1, pairs.astype(jnp.float32), 0.0))
    part_v[...] = part
    pltpu.sync_copy(part_v, out_hbm.at[wid])


@jax.jit
def _pairwise_loss_sc(scores, labels, props):
    mesh = plsc.VectorSubcoreMesh(core_axis_name="c", subcore_axis_name="s")
    scratch = [
        pltpu.VMEM((N,), jnp.float32),
        pltpu.VMEM((N,), jnp.int32),
        pltpu.VMEM((N,), jnp.int32),
        pltpu.VMEM((L,), jnp.float32),
    ] + [pltpu.VMEM((BUF,), jnp.float32) for _ in range(2 * PPW)]
    parts = pl.kernel(
        _sc_body,
        out_type=jax.ShapeDtypeStruct((NW, L), jnp.float32),
        mesh=mesh,
        scratch_types=scratch,
        compiler_params=pltpu.CompilerParams(needs_layout_passes=False),
    )(scores, labels, props)
    loss = parts[:, 0].sum()
    pairs = parts[:, 1].sum()
    return jnp.where(pairs == 0.0, 0.0, loss / jnp.maximum(pairs, 1.0))


def kernel(scores, labels, property_ids):
    scores = scores.reshape(-1).astype(jnp.float32)
    labels = labels.reshape(-1).astype(jnp.int32)
    props = property_ids.reshape(-1).astype(jnp.int32)
    return _pairwise_loss_sc(scores, labels, props)
